# baseline (speedup 1.0000x reference)
import functools

import jax
import jax.numpy as jnp
from jax.experimental import pallas as pl
from jax.experimental.pallas import tpu as pltpu

NE = 64
TOPK = 8
HID = 4096
LBW = 0.01
CH = 128
BT = 1024
NBUF = 2


def _gate_kernel(ntok, x_hbm, w_ref, ids_ref, gw_ref, aux_ref,
                 xbuf, cnt_acc, p_acc, sem):
    nb = ntok // BT
    w = w_ref[...]
    cnt_acc[...] = jnp.zeros_like(cnt_acc)
    p_acc[...] = jnp.zeros_like(p_acc)

    pltpu.make_async_copy(
        x_hbm.at[pl.ds(0, BT)], xbuf.at[0], sem.at[0]).start()

    def step_fn(i, carry):
        slot = jax.lax.rem(i, NBUF)
        nxt = jax.lax.rem(i + 1, NBUF)

        @pl.when(i + 1 < nb)
        def _prefetch():
            pltpu.make_async_copy(
                x_hbm.at[pl.ds((i + 1) * BT, BT)], xbuf.at[nxt],
                sem.at[nxt]).start()

        pltpu.make_async_copy(
            x_hbm.at[pl.ds(i * BT, BT)], xbuf.at[slot], sem.at[slot]).wait()

        x = xbuf[slot]
        logits = jax.lax.dot_general(
            w, x, (((1,), (1,)), ((), ())),
            preferred_element_type=jnp.float32)

        iota = jax.lax.broadcasted_iota(jnp.int32, (NE, CH), 0)
        for c in range(BT // CH):
            lg = jax.lax.slice(logits, (0, c * CH), (NE, (c + 1) * CH))
            m = jnp.max(lg, axis=0, keepdims=True)
            e = jnp.exp(lg - m)
            s = jnp.sum(e, axis=0, keepdims=True)
            pr = e / s

            p_acc[:, c * CH:(c + 1) * CH] += pr

            running = pr
            rows_id, rows_w = [], []
            mx = None
            for _ in range(TOPK):
                mx = jnp.max(running, axis=0, keepdims=True)
                cand = jnp.where(running == mx, iota, NE)
                sel = jnp.min(cand, axis=0, keepdims=True)
                rows_id.append(sel)
                rows_w.append(mx)
                running = jnp.where(iota == sel, -1.0, running)

            cnt_acc[:, c * CH:(c + 1) * CH] += (pr >= mx).astype(jnp.float32)

            ids = jnp.concatenate(rows_id, axis=0)
            ws = jnp.concatenate(rows_w, axis=0)
            wsum = jnp.sum(ws, axis=0, keepdims=True) + 1e-9
            ids_ref[:, pl.ds(i * BT + c * CH, CH)] = ids
            gw_ref[:, pl.ds(i * BT + c * CH, CH)] = ws / wsum
        return carry

    jax.lax.fori_loop(0, nb, step_fn, 0)

    counts = jnp.sum(cnt_acc[...], axis=1, keepdims=True)
    psum = jnp.sum(p_acc[...], axis=1, keepdims=True)
    f = counts / (ntok * TOPK)
    p_mean = psum / ntok
    aux_ref[...] = LBW * NE * jnp.sum(f * p_mean, axis=0, keepdims=True)


def _router(x, W, interpret=False):
    T = x.shape[0]
    return pl.pallas_call(
        functools.partial(_gate_kernel, T),
        in_specs=[
            pl.BlockSpec(memory_space=pl.ANY),
            pl.BlockSpec(memory_space=pltpu.MemorySpace.VMEM),
        ],
        out_specs=[
            pl.BlockSpec(memory_space=pltpu.MemorySpace.VMEM),
            pl.BlockSpec(memory_space=pltpu.MemorySpace.VMEM),
            pl.BlockSpec(memory_space=pltpu.MemorySpace.VMEM),
        ],
        out_shape=[
            jax.ShapeDtypeStruct((TOPK, T), jnp.int32),
            jax.ShapeDtypeStruct((TOPK, T), jnp.float32),
            jax.ShapeDtypeStruct((1, 1), jnp.float32),
        ],
        scratch_shapes=[
            pltpu.VMEM((NBUF, BT, HID), jnp.float32),
            pltpu.VMEM((NE, BT), jnp.float32),
            pltpu.VMEM((NE, BT), jnp.float32),
            pltpu.SemaphoreType.DMA((NBUF,)),
        ],
        interpret=interpret,
    )(x, W)


def kernel(hidden_states, W):
    x = hidden_states.reshape(-1, HID)
    T = x.shape[0]
    ids_t, gw_t, aux = _router(x, W)
    expert_ids = ids_t.T.reshape(-1)
    gate_weights = gw_t.T.reshape(-1)
    token_indices = jax.lax.broadcasted_iota(jnp.int32, (T, TOPK), 0).reshape(-1)
    return expert_ids, gate_weights, token_indices, aux[0, 0]


# speedup vs baseline: 1.0254x; 1.0254x over previous
import functools

import jax
import jax.numpy as jnp
from jax.experimental import pallas as pl
from jax.experimental.pallas import tpu as pltpu

NE = 64
TOPK = 8
HID = 4096
LBW = 0.01
CH = 128
BT = 1024
NBUF = 2


def _gate_kernel(ntok, x_hbm, w_ref, ids_ref, gw_ref, aux_ref,
                 xbuf, cnt_acc, p_acc, sem):
    nb = ntok // BT
    w = w_ref[...]
    cnt_acc[...] = jnp.zeros_like(cnt_acc)
    p_acc[...] = jnp.zeros_like(p_acc)

    pltpu.make_async_copy(
        x_hbm.at[pl.ds(0, BT)], xbuf.at[0], sem.at[0]).start()

    def step_fn(i, carry):
        slot = jax.lax.rem(i, NBUF)
        nxt = jax.lax.rem(i + 1, NBUF)

        @pl.when(i + 1 < nb)
        def _prefetch():
            pltpu.make_async_copy(
                x_hbm.at[pl.ds((i + 1) * BT, BT)], xbuf.at[nxt],
                sem.at[nxt]).start()

        pltpu.make_async_copy(
            x_hbm.at[pl.ds(i * BT, BT)], xbuf.at[slot], sem.at[slot]).wait()

        x = xbuf[slot]
        logits = jax.lax.dot_general(
            w, x, (((1,), (1,)), ((), ())),
            preferred_element_type=jnp.float32)

        iota = jax.lax.broadcasted_iota(jnp.int32, (NE, CH), 0)
        for c in range(BT // CH):
            lg = jax.lax.slice(logits, (0, c * CH), (NE, (c + 1) * CH))
            m = jnp.max(lg, axis=0, keepdims=True)
            e = jnp.exp(lg - m)
            s = jnp.sum(e, axis=0, keepdims=True)
            pr = e / s

            p_acc[:, c * CH:(c + 1) * CH] += pr

            running = pr
            rows_id, rows_w = [], []
            mx = None
            for _ in range(TOPK):
                mx = jnp.max(running, axis=0, keepdims=True)
                cand = jnp.where(running == mx, iota, NE)
                sel = jnp.min(cand, axis=0, keepdims=True)
                rows_id.append(sel)
                rows_w.append(mx)
                running = jnp.where(iota == sel, -1.0, running)

            cnt_acc[:, c * CH:(c + 1) * CH] += (pr >= mx).astype(jnp.float32)

            ids = jnp.concatenate(rows_id, axis=0)
            ws = jnp.concatenate(rows_w, axis=0)
            wsum = jnp.sum(ws, axis=0, keepdims=True) + 1e-9
            ids_ref[pl.ds(i * BT + c * CH, CH), :] = jax.lax.transpose(ids, (1, 0))
            gw_ref[pl.ds(i * BT + c * CH, CH), :] = jax.lax.transpose(ws / wsum, (1, 0))
        return carry

    jax.lax.fori_loop(0, nb, step_fn, 0)

    counts = jnp.sum(cnt_acc[...], axis=1, keepdims=True)
    psum = jnp.sum(p_acc[...], axis=1, keepdims=True)
    f = counts / (ntok * TOPK)
    p_mean = psum / ntok
    aux_ref[...] = LBW * NE * jnp.sum(f * p_mean, axis=0, keepdims=True)


def _router(x, W, interpret=False):
    T = x.shape[0]
    return pl.pallas_call(
        functools.partial(_gate_kernel, T),
        in_specs=[
            pl.BlockSpec(memory_space=pl.ANY),
            pl.BlockSpec(memory_space=pltpu.MemorySpace.VMEM),
        ],
        out_specs=[
            pl.BlockSpec(memory_space=pltpu.MemorySpace.VMEM),
            pl.BlockSpec(memory_space=pltpu.MemorySpace.VMEM),
            pl.BlockSpec(memory_space=pltpu.MemorySpace.VMEM),
        ],
        out_shape=[
            jax.ShapeDtypeStruct((T, TOPK), jnp.int32),
            jax.ShapeDtypeStruct((T, TOPK), jnp.float32),
            jax.ShapeDtypeStruct((1, 1), jnp.float32),
        ],
        scratch_shapes=[
            pltpu.VMEM((NBUF, BT, HID), jnp.float32),
            pltpu.VMEM((NE, BT), jnp.float32),
            pltpu.VMEM((NE, BT), jnp.float32),
            pltpu.SemaphoreType.DMA((NBUF,)),
        ],
        interpret=interpret,
    )(x, W)


def kernel(hidden_states, W):
    x = hidden_states.reshape(-1, HID)
    T = x.shape[0]
    ids_o, gw_o, aux = _router(x, W)
    expert_ids = ids_o.reshape(-1)
    gate_weights = gw_o.reshape(-1)
    token_indices = jax.lax.broadcasted_iota(jnp.int32, (T, TOPK), 0).reshape(-1)
    return expert_ids, gate_weights, token_indices, aux[0, 0]


# speedup vs baseline: 1.0512x; 1.0252x over previous
import functools

import jax
import jax.numpy as jnp
from jax.experimental import pallas as pl
from jax.experimental.pallas import tpu as pltpu

NE = 64
TOPK = 8
HID = 4096
LBW = 0.01
CH = 128
BT = 1024
NBUF = 2


def _gate_kernel(ntok, x_hbm, w_ref, ids_hbm, gw_hbm, aux_ref,
                 xbuf, ids_st, gw_st, cnt_acc, p_acc, sem, osem):
    nb = ntok // BT
    w = w_ref[...]
    cnt_acc[...] = jnp.zeros_like(cnt_acc)
    p_acc[...] = jnp.zeros_like(p_acc)

    pltpu.make_async_copy(
        x_hbm.at[pl.ds(0, BT)], xbuf.at[0], sem.at[0]).start()

    def step_fn(i, carry):
        slot = jax.lax.rem(i, NBUF)
        nxt = jax.lax.rem(i + 1, NBUF)

        @pl.when(i + 1 < nb)
        def _prefetch():
            pltpu.make_async_copy(
                x_hbm.at[pl.ds((i + 1) * BT, BT)], xbuf.at[nxt],
                sem.at[nxt]).start()

        pltpu.make_async_copy(
            x_hbm.at[pl.ds(i * BT, BT)], xbuf.at[slot], sem.at[slot]).wait()

        @pl.when(i >= NBUF)
        def _drain():
            pltpu.make_async_copy(
                ids_st.at[slot], ids_hbm.at[pl.ds((i - NBUF) * BT, BT)],
                osem.at[0, slot]).wait()
            pltpu.make_async_copy(
                gw_st.at[slot], gw_hbm.at[pl.ds((i - NBUF) * BT, BT)],
                osem.at[1, slot]).wait()

        x = xbuf[slot]
        logits = jax.lax.dot_general(
            w, x, (((1,), (1,)), ((), ())),
            preferred_element_type=jnp.float32)

        iota = jax.lax.broadcasted_iota(jnp.int32, (NE, CH), 0)
        for c in range(BT // CH):
            lg = jax.lax.slice(logits, (0, c * CH), (NE, (c + 1) * CH))
            m = jnp.max(lg, axis=0, keepdims=True)
            e = jnp.exp(lg - m)
            s = jnp.sum(e, axis=0, keepdims=True)
            pr = e / s

            p_acc[:, c * CH:(c + 1) * CH] += pr

            running = pr
            rows_id, rows_w = [], []
            mx = None
            for _ in range(TOPK):
                mx = jnp.max(running, axis=0, keepdims=True)
                cand = jnp.where(running == mx, iota, NE)
                sel = jnp.min(cand, axis=0, keepdims=True)
                rows_id.append(sel)
                rows_w.append(mx)
                running = jnp.where(iota == sel, -1.0, running)

            cnt_acc[:, c * CH:(c + 1) * CH] += (pr >= mx).astype(jnp.float32)

            ids = jnp.concatenate(rows_id, axis=0)
            ws = jnp.concatenate(rows_w, axis=0)
            wsum = jnp.sum(ws, axis=0, keepdims=True) + 1e-9
            ids_st[slot, c * CH:(c + 1) * CH, :] = jax.lax.transpose(ids, (1, 0))
            gw_st[slot, c * CH:(c + 1) * CH, :] = jax.lax.transpose(ws / wsum, (1, 0))

        pltpu.make_async_copy(
            ids_st.at[slot], ids_hbm.at[pl.ds(i * BT, BT)],
            osem.at[0, slot]).start()
        pltpu.make_async_copy(
            gw_st.at[slot], gw_hbm.at[pl.ds(i * BT, BT)],
            osem.at[1, slot]).start()
        return carry

    jax.lax.fori_loop(0, nb, step_fn, 0)

    for j in range(NBUF):
        i = nb - NBUF + j
        if i >= 0:
            slot = i % NBUF
            pltpu.make_async_copy(
                ids_st.at[slot], ids_hbm.at[pl.ds(i * BT, BT)],
                osem.at[0, slot]).wait()
            pltpu.make_async_copy(
                gw_st.at[slot], gw_hbm.at[pl.ds(i * BT, BT)],
                osem.at[1, slot]).wait()

    counts = jnp.sum(cnt_acc[...], axis=1, keepdims=True)
    psum = jnp.sum(p_acc[...], axis=1, keepdims=True)
    f = counts / (ntok * TOPK)
    p_mean = psum / ntok
    aux_ref[...] = LBW * NE * jnp.sum(f * p_mean, axis=0, keepdims=True)


def _router(x, W, interpret=False):
    T = x.shape[0]
    return pl.pallas_call(
        functools.partial(_gate_kernel, T),
        in_specs=[
            pl.BlockSpec(memory_space=pl.ANY),
            pl.BlockSpec(memory_space=pltpu.MemorySpace.VMEM),
        ],
        out_specs=[
            pl.BlockSpec(memory_space=pl.ANY),
            pl.BlockSpec(memory_space=pl.ANY),
            pl.BlockSpec(memory_space=pltpu.MemorySpace.VMEM),
        ],
        out_shape=[
            jax.ShapeDtypeStruct((T, TOPK), jnp.int32),
            jax.ShapeDtypeStruct((T, TOPK), jnp.float32),
            jax.ShapeDtypeStruct((1, 1), jnp.float32),
        ],
        scratch_shapes=[
            pltpu.VMEM((NBUF, BT, HID), jnp.float32),
            pltpu.VMEM((NBUF, BT, TOPK), jnp.int32),
            pltpu.VMEM((NBUF, BT, TOPK), jnp.float32),
            pltpu.VMEM((NE, BT), jnp.float32),
            pltpu.VMEM((NE, BT), jnp.float32),
            pltpu.SemaphoreType.DMA((NBUF,)),
            pltpu.SemaphoreType.DMA((2, NBUF)),
        ],
        interpret=interpret,
    )(x, W)


def kernel(hidden_states, W):
    x = hidden_states.reshape(-1, HID)
    T = x.shape[0]
    ids_o, gw_o, aux = _router(x, W)
    expert_ids = ids_o.reshape(-1)
    gate_weights = gw_o.reshape(-1)
    token_indices = jax.lax.broadcasted_iota(jnp.int32, (T, TOPK), 0).reshape(-1)
    return expert_ids, gate_weights, token_indices, aux[0, 0]


# speedup vs baseline: 1.0534x; 1.0021x over previous
import functools

import jax
import jax.numpy as jnp
from jax.experimental import pallas as pl
from jax.experimental.pallas import tpu as pltpu

NE = 64
TOPK = 8
HID = 4096
LBW = 0.01
CH = 128
BT = 1024
NBUF = 2


def _gate_kernel(ntok, x_hbm, w_ref, ids_hbm, gw_hbm, aux_ref,
                 xbuf, ids_st, gw_st, cnt_acc, p_acc, sem, osem):
    nb = ntok // BT
    w = w_ref[...]
    cnt_acc[...] = jnp.zeros_like(cnt_acc)
    p_acc[...] = jnp.zeros_like(p_acc)

    pltpu.make_async_copy(
        x_hbm.at[pl.ds(0, BT)], xbuf.at[0], sem.at[0]).start()

    def step_fn(i, carry):
        slot = jax.lax.rem(i, NBUF)
        nxt = jax.lax.rem(i + 1, NBUF)

        @pl.when(i + 1 < nb)
        def _prefetch():
            pltpu.make_async_copy(
                x_hbm.at[pl.ds((i + 1) * BT, BT)], xbuf.at[nxt],
                sem.at[nxt]).start()

        pltpu.make_async_copy(
            x_hbm.at[pl.ds(i * BT, BT)], xbuf.at[slot], sem.at[slot]).wait()

        @pl.when(i >= NBUF)
        def _drain():
            pltpu.make_async_copy(
                ids_st.at[slot], ids_hbm.at[pl.ds((i - NBUF) * BT, BT)],
                osem.at[0, slot]).wait()
            pltpu.make_async_copy(
                gw_st.at[slot], gw_hbm.at[pl.ds((i - NBUF) * BT, BT)],
                osem.at[1, slot]).wait()

        x = xbuf[slot]
        logits = jax.lax.dot_general(
            w, x, (((1,), (1,)), ((), ())),
            preferred_element_type=jnp.float32)

        revidx = (NE - 1) - jax.lax.broadcasted_iota(jnp.int32, (NE, CH), 0)
        for c in range(BT // CH):
            lg = jax.lax.slice(logits, (0, c * CH), (NE, (c + 1) * CH))
            m = jnp.max(lg, axis=0, keepdims=True)
            e = jnp.exp(lg - m)
            s = jnp.sum(e, axis=0, keepdims=True)
            pr = e / s

            p_acc[:, c * CH:(c + 1) * CH] += pr

            pk = (jax.lax.bitcast_convert_type(pr + 1.0, jnp.int32)
                  - jnp.int32(0x3F800000))
            keys0 = (pk << 6) | revidx
            keys = keys0
            rows_id, rows_k = [], []
            mxk = None
            for _ in range(TOPK):
                mxk = jnp.max(keys, axis=0, keepdims=True)
                rows_id.append((NE - 1) - (mxk & (NE - 1)))
                rows_k.append(mxk)
                keys = jnp.where(keys == mxk, 0, keys)

            cnt_acc[:, c * CH:(c + 1) * CH] += (keys0 >= mxk).astype(jnp.float32)

            ids = jnp.concatenate(rows_id, axis=0)
            ws = jax.lax.bitcast_convert_type(
                (jnp.concatenate(rows_k, axis=0) >> 6)
                + jnp.int32(0x3F800000), jnp.float32) - 1.0
            wsum = jnp.sum(ws, axis=0, keepdims=True) + 1e-9
            ids_st[slot, c * CH:(c + 1) * CH, :] = jax.lax.transpose(ids, (1, 0))
            gw_st[slot, c * CH:(c + 1) * CH, :] = jax.lax.transpose(ws / wsum, (1, 0))

        pltpu.make_async_copy(
            ids_st.at[slot], ids_hbm.at[pl.ds(i * BT, BT)],
            osem.at[0, slot]).start()
        pltpu.make_async_copy(
            gw_st.at[slot], gw_hbm.at[pl.ds(i * BT, BT)],
            osem.at[1, slot]).start()
        return carry

    jax.lax.fori_loop(0, nb, step_fn, 0)

    for j in range(NBUF):
        i = nb - NBUF + j
        if i >= 0:
            slot = i % NBUF
            pltpu.make_async_copy(
                ids_st.at[slot], ids_hbm.at[pl.ds(i * BT, BT)],
                osem.at[0, slot]).wait()
            pltpu.make_async_copy(
                gw_st.at[slot], gw_hbm.at[pl.ds(i * BT, BT)],
                osem.at[1, slot]).wait()

    counts = jnp.sum(cnt_acc[...], axis=1, keepdims=True)
    psum = jnp.sum(p_acc[...], axis=1, keepdims=True)
    f = counts / (ntok * TOPK)
    p_mean = psum / ntok
    aux_ref[...] = LBW * NE * jnp.sum(f * p_mean, axis=0, keepdims=True)


def _router(x, W, interpret=False):
    T = x.shape[0]
    return pl.pallas_call(
        functools.partial(_gate_kernel, T),
        in_specs=[
            pl.BlockSpec(memory_space=pl.ANY),
            pl.BlockSpec(memory_space=pltpu.MemorySpace.VMEM),
        ],
        out_specs=[
            pl.BlockSpec(memory_space=pl.ANY),
            pl.BlockSpec(memory_space=pl.ANY),
            pl.BlockSpec(memory_space=pltpu.MemorySpace.VMEM),
        ],
        out_shape=[
            jax.ShapeDtypeStruct((T, TOPK), jnp.int32),
            jax.ShapeDtypeStruct((T, TOPK), jnp.float32),
            jax.ShapeDtypeStruct((1, 1), jnp.float32),
        ],
        scratch_shapes=[
            pltpu.VMEM((NBUF, BT, HID), jnp.float32),
            pltpu.VMEM((NBUF, BT, TOPK), jnp.int32),
            pltpu.VMEM((NBUF, BT, TOPK), jnp.float32),
            pltpu.VMEM((NE, BT), jnp.float32),
            pltpu.VMEM((NE, BT), jnp.float32),
            pltpu.SemaphoreType.DMA((NBUF,)),
            pltpu.SemaphoreType.DMA((2, NBUF)),
        ],
        interpret=interpret,
    )(x, W)


def kernel(hidden_states, W):
    x = hidden_states.reshape(-1, HID)
    T = x.shape[0]
    ids_o, gw_o, aux = _router(x, W)
    expert_ids = ids_o.reshape(-1)
    gate_weights = gw_o.reshape(-1)
    token_indices = jax.lax.broadcasted_iota(jnp.int32, (T, TOPK), 0).reshape(-1)
    return expert_ids, gate_weights, token_indices, aux[0, 0]
